# Initial kernel scaffold; baseline (speedup 1.0000x reference)
import jax
import jax.numpy as jnp
from jax import lax
from jax.experimental import pallas as pl
from jax.experimental.pallas import tpu as pltpu
from jax.experimental.pallas import tpu_sc as plsc

N = 10000
E = 160000
D = 256
H = 128
R = 1000
NT = 16
STRIPE = 640
LAST = N - 15 * STRIPE
EC = 80
ECI = E // NT // EC
DC = 40
DCI = (E // 2) // NT // DC

_sc_mesh = plsc.VectorSubcoreMesh(core_axis_name="c", subcore_axis_name="s")


def _stripe_copy(src_ref, dst_ref, s):
    @pl.when(s < 15)
    def _():
        pltpu.sync_copy(src_ref.at[pl.ds(s * STRIPE, STRIPE)],
                        dst_ref.at[pl.ds(s * STRIPE, STRIPE)])

    @pl.when(s == 15)
    def _():
        pltpu.sync_copy(src_ref.at[pl.ds(15 * STRIPE, LAST)],
                        dst_ref.at[pl.ds(15 * STRIPE, LAST)])


def _deg_body(dst_hbm, dga, dgb, idx_d, ones_v, zbuf, acc):
    c = lax.axis_index("c")
    s = lax.axis_index("s")
    for off in (0, 16, 24):
        ones_v[pl.ds(off, 16)] = jnp.ones((16,), jnp.float32)

    def _zb(k, carry):
        zbuf[pl.ds(k * 16, 16)] = jnp.zeros((16,), jnp.float32)
        return carry

    lax.fori_loop(0, STRIPE // 16, _zb, 0)
    _stripe_copy(zbuf, acc, s)
    plsc.subcore_barrier()

    for cc, out in ((0, dga), (1, dgb)):
        @pl.when(c == cc)
        def _(cc=cc, out=out):
            def _body(i, carry):
                base = cc * (E // 2) + s * (E // 2 // NT) + i * DC
                pltpu.sync_copy(dst_hbm.at[pl.ds(base, DC)], idx_d)
                pltpu.sync_copy(ones_v, acc.at[idx_d], add=True)
                return carry

            lax.fori_loop(0, DCI, _body, 0)
            plsc.subcore_barrier()
            _stripe_copy(acc, out, s)


_deg = pl.kernel(
    _deg_body,
    out_type=[jax.ShapeDtypeStruct((N,), jnp.float32),
              jax.ShapeDtypeStruct((N,), jnp.float32)],
    mesh=_sc_mesh,
    scratch_types=[
        pltpu.VMEM((DC,), jnp.int32),
        pltpu.VMEM((DC,), jnp.float32),
        pltpu.VMEM((STRIPE,), jnp.float32),
        pltpu.VMEM_SHARED((N,), jnp.float32),
    ],
)


def _edge_body(yp0, yp1, src_hbm, dst_hbm, z0, z1, idx_s, idx_d, rows, acc, sem):
    c = lax.axis_index("c")
    s = lax.axis_index("s")
    for cc, yp, z in ((0, yp0, z0), (1, yp1, z1)):
        @pl.when(c == cc)
        def _(yp=yp, z=z):
            _stripe_copy(yp, acc, s)
            plsc.subcore_barrier()

            def _body(i, carry):
                base = s * (E // NT) + i * EC
                pltpu.sync_copy(src_hbm.at[pl.ds(base, EC)], idx_s)
                pltpu.sync_copy(dst_hbm.at[pl.ds(base, EC)], idx_d)
                pltpu.async_copy(yp.at[idx_s], rows, sem).wait()
                pltpu.sync_copy(rows, acc.at[idx_d], add=True)
                return carry

            lax.fori_loop(0, ECI, _body, 0)
            plsc.subcore_barrier()
            _stripe_copy(acc, z, s)


_edge = pl.kernel(
    _edge_body,
    out_type=[jax.ShapeDtypeStruct((N, H), jnp.float32),
              jax.ShapeDtypeStruct((N, H), jnp.float32)],
    mesh=_sc_mesh,
    scratch_types=[
        pltpu.VMEM((EC,), jnp.int32),
        pltpu.VMEM((EC,), jnp.int32),
        pltpu.VMEM((EC, H), jnp.float32),
        pltpu.VMEM_SHARED((N, H), jnp.float32),
        pltpu.SemaphoreType.DMA,
    ],
)


def _tc1_body(x_ref, w_ref, da_ref, db_ref, y0_ref, y1_ref, u_ref):
    u = lax.rsqrt(da_ref[...] + db_ref[...] + 1.0)
    y = jnp.dot(x_ref[...], w_ref[...], preferred_element_type=jnp.float32) * u
    y0_ref[...] = y[:, :H]
    y1_ref[...] = y[:, H:]
    u_ref[...] = u


_tc1 = pl.pallas_call(
    _tc1_body,
    grid=(N // R,),
    in_specs=[
        pl.BlockSpec((R, D), lambda i: (i, 0)),
        pl.BlockSpec((D, D), lambda i: (0, 0)),
        pl.BlockSpec((R, 1), lambda i: (i, 0)),
        pl.BlockSpec((R, 1), lambda i: (i, 0)),
    ],
    out_specs=[
        pl.BlockSpec((R, H), lambda i: (i, 0)),
        pl.BlockSpec((R, H), lambda i: (i, 0)),
        pl.BlockSpec((R, 1), lambda i: (i, 0)),
    ],
    out_shape=[
        jax.ShapeDtypeStruct((N, H), jnp.float32),
        jax.ShapeDtypeStruct((N, H), jnp.float32),
        jax.ShapeDtypeStruct((N, 1), jnp.float32),
    ],
)


def _tc2_body(z0_ref, z1_ref, u_ref, b_ref, w_ref, y0_ref, y1_ref):
    u = u_ref[...]
    h0 = jnp.maximum(z0_ref[...] * u + b_ref[:, :H], 0.0)
    h1 = jnp.maximum(z1_ref[...] * u + b_ref[:, H:], 0.0)
    h = jnp.concatenate([h0, h1], axis=1)
    y = jnp.dot(h, w_ref[...], preferred_element_type=jnp.float32) * u
    y0_ref[...] = y[:, :H]
    y1_ref[...] = y[:, H:]


_tc2 = pl.pallas_call(
    _tc2_body,
    grid=(N // R,),
    in_specs=[
        pl.BlockSpec((R, H), lambda i: (i, 0)),
        pl.BlockSpec((R, H), lambda i: (i, 0)),
        pl.BlockSpec((R, 1), lambda i: (i, 0)),
        pl.BlockSpec((1, D), lambda i: (0, 0)),
        pl.BlockSpec((D, D), lambda i: (0, 0)),
    ],
    out_specs=[
        pl.BlockSpec((R, H), lambda i: (i, 0)),
        pl.BlockSpec((R, H), lambda i: (i, 0)),
    ],
    out_shape=[
        jax.ShapeDtypeStruct((N, H), jnp.float32),
        jax.ShapeDtypeStruct((N, H), jnp.float32),
    ],
)


def _tc3_body(z0_ref, z1_ref, u_ref, b_ref, wl1_ref, bl1_ref, wl2_ref, bl2_ref,
              o_ref, acc_ref):
    i = pl.program_id(0)

    @pl.when(i == 0)
    def _():
        acc_ref[...] = jnp.zeros_like(acc_ref)

    u = u_ref[...]
    h0 = jnp.maximum(z0_ref[...] * u + b_ref[:, :H], 0.0)
    h1 = jnp.maximum(z1_ref[...] * u + b_ref[:, H:], 0.0)
    acc_ref[:, :H] += jnp.sum(h0, axis=0, keepdims=True)
    acc_ref[:, H:] += jnp.sum(h1, axis=0, keepdims=True)

    @pl.when(i == pl.num_programs(0) - 1)
    def _():
        g = acc_ref[...] * (1.0 / N)
        t = jnp.maximum(
            jnp.dot(g, wl1_ref[...], preferred_element_type=jnp.float32)
            + bl1_ref[...], 0.0)
        o = jnp.maximum(
            jnp.dot(t, wl2_ref[...], preferred_element_type=jnp.float32)
            + bl2_ref[...], 0.0)
        o_ref[...] = o


_tc3 = pl.pallas_call(
    _tc3_body,
    grid=(N // R,),
    in_specs=[
        pl.BlockSpec((R, H), lambda i: (i, 0)),
        pl.BlockSpec((R, H), lambda i: (i, 0)),
        pl.BlockSpec((R, 1), lambda i: (i, 0)),
        pl.BlockSpec((1, D), lambda i: (0, 0)),
        pl.BlockSpec((D, D), lambda i: (0, 0)),
        pl.BlockSpec((1, D), lambda i: (0, 0)),
        pl.BlockSpec((D, 1), lambda i: (0, 0)),
        pl.BlockSpec((1, 1), lambda i: (0, 0)),
    ],
    out_specs=pl.BlockSpec((1, 1), lambda i: (0, 0)),
    out_shape=jax.ShapeDtypeStruct((1, 1), jnp.float32),
    scratch_shapes=[pltpu.VMEM((1, D), jnp.float32)],
)


def kernel(x, edge_index, W1, b1, W2, b2, Wl1, bl1, Wl2, bl2):
    src = edge_index[0]
    dst = edge_index[1]
    dga, dgb = _deg(dst)
    y0, y1, u = _tc1(x, W1, dga.reshape(N, 1), dgb.reshape(N, 1))
    z0, z1 = _edge(y0, y1, src, dst)
    y0, y1 = _tc2(z0, z1, u, b1.reshape(1, D), W2)
    z0, z1 = _edge(y0, y1, src, dst)
    return _tc3(z0, z1, u, b2.reshape(1, D), Wl1, bl1.reshape(1, D),
                Wl2, bl2.reshape(1, 1))


# speedup vs baseline: 7.9190x; 7.9190x over previous
import jax
import jax.numpy as jnp
from jax import lax
from jax.experimental import pallas as pl
from jax.experimental.pallas import tpu as pltpu
from jax.experimental.pallas import tpu_sc as plsc

N = 10000
NP = 10240
E = 160000
D = 256
H = 128
R = 1000
NT = 16
STRIPE = NP // NT
EC = 80
ECI = E // NT // EC
DC = 40
DCI = (E // 2) // NT // DC

_sc_mesh = plsc.VectorSubcoreMesh(core_axis_name="c", subcore_axis_name="s")


def _stripe_copy(src_ref, dst_ref, s):
    pltpu.sync_copy(src_ref.at[pl.ds(s * STRIPE, STRIPE)],
                    dst_ref.at[pl.ds(s * STRIPE, STRIPE)])


def _deg_body(dst_hbm, dga, dgb, idx_d, ones_v, zbuf, acc):
    c = lax.axis_index("c")
    s = lax.axis_index("s")
    for off in (0, 16, 24):
        ones_v[pl.ds(off, 16)] = jnp.ones((16,), jnp.float32)

    def _zb(k, carry):
        zbuf[pl.ds(k * 16, 16)] = jnp.zeros((16,), jnp.float32)
        return carry

    lax.fori_loop(0, STRIPE // 16, _zb, 0)

    pltpu.sync_copy(zbuf, acc.at[pl.ds(s * STRIPE, STRIPE)])
    plsc.subcore_barrier()

    for cc, out in ((0, dga), (1, dgb)):
        @pl.when(c == cc)
        def _(cc=cc, out=out):
            def _body(i, carry):
                base = cc * (E // 2) + s * (E // 2 // NT) + i * DC
                pltpu.sync_copy(dst_hbm.at[pl.ds(base, DC)], idx_d)
                pltpu.sync_copy(ones_v, acc.at[idx_d], add=True)
                return carry

            lax.fori_loop(0, DCI, _body, 0)
            plsc.subcore_barrier()
            _stripe_copy(acc, out, s)


_deg = pl.kernel(
    _deg_body,
    out_type=[jax.ShapeDtypeStruct((NP,), jnp.float32),
              jax.ShapeDtypeStruct((NP,), jnp.float32)],
    mesh=_sc_mesh,
    scratch_types=[
        pltpu.VMEM((DC,), jnp.int32),
        pltpu.VMEM((DC,), jnp.float32),
        pltpu.VMEM((STRIPE,), jnp.float32),
        pltpu.VMEM_SHARED((NP,), jnp.float32),
    ],
)


def _edge_body(yp0, yp1, src_hbm, dst_hbm, z0, z1, idx_s, idx_d, rows, acc, sem):
    c = lax.axis_index("c")
    s = lax.axis_index("s")
    for cc, yp, z in ((0, yp0, z0), (1, yp1, z1)):
        @pl.when(c == cc)
        def _(yp=yp, z=z):
            _stripe_copy(yp, acc, s)
            plsc.subcore_barrier()

            def _body(i, carry):
                base = s * (E // NT) + i * EC
                pltpu.sync_copy(src_hbm.at[pl.ds(base, EC)], idx_s)
                pltpu.sync_copy(dst_hbm.at[pl.ds(base, EC)], idx_d)
                pltpu.async_copy(yp.at[idx_s], rows, sem).wait()
                pltpu.sync_copy(rows, acc.at[idx_d], add=True)
                return carry

            lax.fori_loop(0, ECI, _body, 0)
            plsc.subcore_barrier()
            _stripe_copy(acc, z, s)


_edge = pl.kernel(
    _edge_body,
    out_type=[jax.ShapeDtypeStruct((NP, H), jnp.float32),
              jax.ShapeDtypeStruct((NP, H), jnp.float32)],
    mesh=_sc_mesh,
    scratch_types=[
        pltpu.VMEM((EC,), jnp.int32),
        pltpu.VMEM((EC,), jnp.int32),
        pltpu.VMEM((EC, H), jnp.float32),
        pltpu.VMEM_SHARED((NP, H), jnp.float32),
        pltpu.SemaphoreType.DMA,
    ],
)


def _tc1_body(x_ref, w_ref, da_ref, db_ref, y0_ref, y1_ref, u_ref):
    u = lax.rsqrt(da_ref[...] + db_ref[...] + 1.0)
    y = jnp.dot(x_ref[...], w_ref[...], preferred_element_type=jnp.float32) * u
    y0_ref[...] = y[:, :H]
    y1_ref[...] = y[:, H:]
    u_ref[...] = u


_tc1 = pl.pallas_call(
    _tc1_body,
    grid=(N // R,),
    in_specs=[
        pl.BlockSpec((R, D), lambda i: (i, 0)),
        pl.BlockSpec((D, D), lambda i: (0, 0)),
        pl.BlockSpec((R, 1), lambda i: (i, 0)),
        pl.BlockSpec((R, 1), lambda i: (i, 0)),
    ],
    out_specs=[
        pl.BlockSpec((R, H), lambda i: (i, 0)),
        pl.BlockSpec((R, H), lambda i: (i, 0)),
        pl.BlockSpec((R, 1), lambda i: (i, 0)),
    ],
    out_shape=[
        jax.ShapeDtypeStruct((NP, H), jnp.float32),
        jax.ShapeDtypeStruct((NP, H), jnp.float32),
        jax.ShapeDtypeStruct((N, 1), jnp.float32),
    ],
)


def _tc2_body(z0_ref, z1_ref, u_ref, b_ref, w_ref, y0_ref, y1_ref):
    u = u_ref[...]
    h0 = jnp.maximum(z0_ref[...] * u + b_ref[:, :H], 0.0)
    h1 = jnp.maximum(z1_ref[...] * u + b_ref[:, H:], 0.0)
    h = jnp.concatenate([h0, h1], axis=1)
    y = jnp.dot(h, w_ref[...], preferred_element_type=jnp.float32) * u
    y0_ref[...] = y[:, :H]
    y1_ref[...] = y[:, H:]


_tc2 = pl.pallas_call(
    _tc2_body,
    grid=(N // R,),
    in_specs=[
        pl.BlockSpec((R, H), lambda i: (i, 0)),
        pl.BlockSpec((R, H), lambda i: (i, 0)),
        pl.BlockSpec((R, 1), lambda i: (i, 0)),
        pl.BlockSpec((1, D), lambda i: (0, 0)),
        pl.BlockSpec((D, D), lambda i: (0, 0)),
    ],
    out_specs=[
        pl.BlockSpec((R, H), lambda i: (i, 0)),
        pl.BlockSpec((R, H), lambda i: (i, 0)),
    ],
    out_shape=[
        jax.ShapeDtypeStruct((NP, H), jnp.float32),
        jax.ShapeDtypeStruct((NP, H), jnp.float32),
    ],
)


def _tc3_body(z0_ref, z1_ref, u_ref, b_ref, wl1_ref, bl1_ref, wl2_ref, bl2_ref,
              o_ref, acc_ref):
    i = pl.program_id(0)

    @pl.when(i == 0)
    def _():
        acc_ref[...] = jnp.zeros_like(acc_ref)

    u = u_ref[...]
    h0 = jnp.maximum(z0_ref[...] * u + b_ref[:, :H], 0.0)
    h1 = jnp.maximum(z1_ref[...] * u + b_ref[:, H:], 0.0)
    acc_ref[:, :H] += jnp.sum(h0, axis=0, keepdims=True)
    acc_ref[:, H:] += jnp.sum(h1, axis=0, keepdims=True)

    @pl.when(i == pl.num_programs(0) - 1)
    def _():
        g = acc_ref[...] * (1.0 / N)
        t = jnp.maximum(
            jnp.dot(g, wl1_ref[...], preferred_element_type=jnp.float32)
            + bl1_ref[...], 0.0)
        o = jnp.maximum(
            jnp.dot(t, wl2_ref[...], preferred_element_type=jnp.float32)
            + bl2_ref[...], 0.0)
        o_ref[...] = o


_tc3 = pl.pallas_call(
    _tc3_body,
    grid=(N // R,),
    in_specs=[
        pl.BlockSpec((R, H), lambda i: (i, 0)),
        pl.BlockSpec((R, H), lambda i: (i, 0)),
        pl.BlockSpec((R, 1), lambda i: (i, 0)),
        pl.BlockSpec((1, D), lambda i: (0, 0)),
        pl.BlockSpec((D, D), lambda i: (0, 0)),
        pl.BlockSpec((1, D), lambda i: (0, 0)),
        pl.BlockSpec((D, 1), lambda i: (0, 0)),
        pl.BlockSpec((1, 1), lambda i: (0, 0)),
    ],
    out_specs=pl.BlockSpec((1, 1), lambda i: (0, 0)),
    out_shape=jax.ShapeDtypeStruct((1, 1), jnp.float32),
    scratch_shapes=[pltpu.VMEM((1, D), jnp.float32)],
)


def kernel(x, edge_index, W1, b1, W2, b2, Wl1, bl1, Wl2, bl2):
    src = edge_index[0]
    dst = edge_index[1]
    dga, dgb = _deg(dst)
    y0, y1, u = _tc1(x, W1, dga[:N].reshape(N, 1), dgb[:N].reshape(N, 1))
    z0, z1 = _edge(y0, y1, src, dst)
    y0, y1 = _tc2(z0, z1, u, b1.reshape(1, D), W2)
    z0, z1 = _edge(y0, y1, src, dst)
    return _tc3(z0, z1, u, b2.reshape(1, D), Wl1, bl1.reshape(1, D),
                Wl2, bl2.reshape(1, 1))


# speedup vs baseline: 19.8530x; 2.5070x over previous
import jax
import jax.numpy as jnp
from jax import lax
from jax.experimental import pallas as pl
from jax.experimental.pallas import tpu as pltpu
from jax.experimental.pallas import tpu_sc as plsc

N = 10000
NP = 10240
E = 160000
D = 256
H = 128
R = 1000
NT = 16
STRIPE = NP // NT
EC = 125
NCH = E // EC
CPT = NCH // NT
HB = CPT // 2
DPT = NCH // 2 // NT

_sc_mesh = plsc.VectorSubcoreMesh(core_axis_name="c", subcore_axis_name="s")


def _stripe_copy(src_ref, dst_ref, s):
    pltpu.sync_copy(src_ref.at[pl.ds(s * STRIPE, STRIPE)],
                    dst_ref.at[pl.ds(s * STRIPE, STRIPE)])


def _deg_body(dst2, dga, dgb, idx_d, ones_v, zbuf, acc):
    c = lax.axis_index("c")
    s = lax.axis_index("s")
    for off in range(0, 128, 16):
        ones_v[pl.ds(off, 16)] = jnp.ones((16,), jnp.float32)

    def _zb(k, carry):
        zbuf[pl.ds(k * 16, 16)] = jnp.zeros((16,), jnp.float32)
        return carry

    lax.fori_loop(0, STRIPE // 16, _zb, 0)

    pltpu.sync_copy(zbuf, acc.at[pl.ds(s * STRIPE, STRIPE)])
    plsc.subcore_barrier()

    for cc, out in ((0, dga), (1, dgb)):
        @pl.when(c == cc)
        def _(cc=cc, out=out):
            pltpu.sync_copy(dst2.at[pl.ds(cc * (NCH // 2) + s * DPT, DPT)],
                            idx_d)

            def _body(j, carry):
                pltpu.sync_copy(ones_v.at[pl.ds(0, EC)], acc.at[idx_d.at[j]],
                                add=True)
                return carry

            lax.fori_loop(0, DPT, _body, 0)
            plsc.subcore_barrier()
            _stripe_copy(acc, out, s)


_deg = pl.kernel(
    _deg_body,
    out_type=[jax.ShapeDtypeStruct((NP,), jnp.float32),
              jax.ShapeDtypeStruct((NP,), jnp.float32)],
    mesh=_sc_mesh,
    scratch_types=[
        pltpu.VMEM((DPT, EC), jnp.int32),
        pltpu.VMEM((128,), jnp.float32),
        pltpu.VMEM((STRIPE,), jnp.float32),
        pltpu.VMEM_SHARED((NP,), jnp.float32),
    ],
)


def _edge_body(yp0, yp1, src2, dst2, z0, z1, idx_s, idx_d, buf0, buf1, acc,
               sem0, sem1):
    c = lax.axis_index("c")
    s = lax.axis_index("s")
    for cc, yp, z in ((0, yp0, z0), (1, yp1, z1)):
        @pl.when(c == cc)
        def _(yp=yp, z=z):
            _stripe_copy(yp, acc, s)
            plsc.subcore_barrier()

            for h in range(2):
                rb = s * CPT + h * HB
                pltpu.sync_copy(src2.at[pl.ds(rb, HB)], idx_s)
                pltpu.sync_copy(dst2.at[pl.ds(rb, HB)], idx_d)
                pltpu.async_copy(yp.at[idx_s.at[0]], buf0, sem0)
                pltpu.async_copy(yp.at[idx_s.at[1]], buf1, sem1)

                def _body(k, carry):
                    pltpu.make_async_copy(yp.at[idx_s.at[0]], buf0, sem0).wait()
                    pltpu.sync_copy(buf0, acc.at[idx_d.at[2 * k]], add=True)

                    @pl.when(k < HB // 2 - 1)
                    def _():
                        pltpu.async_copy(yp.at[idx_s.at[2 * k + 2]], buf0, sem0)

                    pltpu.make_async_copy(yp.at[idx_s.at[1]], buf1, sem1).wait()
                    pltpu.sync_copy(buf1, acc.at[idx_d.at[2 * k + 1]], add=True)

                    @pl.when(k < HB // 2 - 1)
                    def _():
                        pltpu.async_copy(yp.at[idx_s.at[2 * k + 3]], buf1, sem1)

                    return carry

                lax.fori_loop(0, HB // 2, _body, 0)

            plsc.subcore_barrier()
            _stripe_copy(acc, z, s)


_edge = pl.kernel(
    _edge_body,
    out_type=[jax.ShapeDtypeStruct((NP, H), jnp.float32),
              jax.ShapeDtypeStruct((NP, H), jnp.float32)],
    mesh=_sc_mesh,
    scratch_types=[
        pltpu.VMEM((HB, EC), jnp.int32),
        pltpu.VMEM((HB, EC), jnp.int32),
        pltpu.VMEM((EC, H), jnp.float32),
        pltpu.VMEM((EC, H), jnp.float32),
        pltpu.VMEM_SHARED((NP, H), jnp.float32),
        pltpu.SemaphoreType.DMA,
        pltpu.SemaphoreType.DMA,
    ],
)


def _tc1_body(x_ref, w_ref, da_ref, db_ref, y0_ref, y1_ref, u_ref):
    u = lax.rsqrt(da_ref[...] + db_ref[...] + 1.0)
    y = jnp.dot(x_ref[...], w_ref[...], preferred_element_type=jnp.float32) * u
    y0_ref[...] = y[:, :H]
    y1_ref[...] = y[:, H:]
    u_ref[...] = u


_tc1 = pl.pallas_call(
    _tc1_body,
    grid=(N // R,),
    in_specs=[
        pl.BlockSpec((R, D), lambda i: (i, 0)),
        pl.BlockSpec((D, D), lambda i: (0, 0)),
        pl.BlockSpec((R, 1), lambda i: (i, 0)),
        pl.BlockSpec((R, 1), lambda i: (i, 0)),
    ],
    out_specs=[
        pl.BlockSpec((R, H), lambda i: (i, 0)),
        pl.BlockSpec((R, H), lambda i: (i, 0)),
        pl.BlockSpec((R, 1), lambda i: (i, 0)),
    ],
    out_shape=[
        jax.ShapeDtypeStruct((NP, H), jnp.float32),
        jax.ShapeDtypeStruct((NP, H), jnp.float32),
        jax.ShapeDtypeStruct((N, 1), jnp.float32),
    ],
)


def _tc2_body(z0_ref, z1_ref, u_ref, b_ref, w_ref, y0_ref, y1_ref):
    u = u_ref[...]
    h0 = jnp.maximum(z0_ref[...] * u + b_ref[:, :H], 0.0)
    h1 = jnp.maximum(z1_ref[...] * u + b_ref[:, H:], 0.0)
    h = jnp.concatenate([h0, h1], axis=1)
    y = jnp.dot(h, w_ref[...], preferred_element_type=jnp.float32) * u
    y0_ref[...] = y[:, :H]
    y1_ref[...] = y[:, H:]


_tc2 = pl.pallas_call(
    _tc2_body,
    grid=(N // R,),
    in_specs=[
        pl.BlockSpec((R, H), lambda i: (i, 0)),
        pl.BlockSpec((R, H), lambda i: (i, 0)),
        pl.BlockSpec((R, 1), lambda i: (i, 0)),
        pl.BlockSpec((1, D), lambda i: (0, 0)),
        pl.BlockSpec((D, D), lambda i: (0, 0)),
    ],
    out_specs=[
        pl.BlockSpec((R, H), lambda i: (i, 0)),
        pl.BlockSpec((R, H), lambda i: (i, 0)),
    ],
    out_shape=[
        jax.ShapeDtypeStruct((NP, H), jnp.float32),
        jax.ShapeDtypeStruct((NP, H), jnp.float32),
    ],
)


def _tc3_body(z0_ref, z1_ref, u_ref, b_ref, wl1_ref, bl1_ref, wl2_ref, bl2_ref,
              o_ref, acc_ref):
    i = pl.program_id(0)

    @pl.when(i == 0)
    def _():
        acc_ref[...] = jnp.zeros_like(acc_ref)

    u = u_ref[...]
    h0 = jnp.maximum(z0_ref[...] * u + b_ref[:, :H], 0.0)
    h1 = jnp.maximum(z1_ref[...] * u + b_ref[:, H:], 0.0)
    acc_ref[:, :H] += jnp.sum(h0, axis=0, keepdims=True)
    acc_ref[:, H:] += jnp.sum(h1, axis=0, keepdims=True)

    @pl.when(i == pl.num_programs(0) - 1)
    def _():
        g = acc_ref[...] * (1.0 / N)
        t = jnp.maximum(
            jnp.dot(g, wl1_ref[...], preferred_element_type=jnp.float32)
            + bl1_ref[...], 0.0)
        o = jnp.maximum(
            jnp.dot(t, wl2_ref[...], preferred_element_type=jnp.float32)
            + bl2_ref[...], 0.0)
        o_ref[...] = o


_tc3 = pl.pallas_call(
    _tc3_body,
    grid=(N // R,),
    in_specs=[
        pl.BlockSpec((R, H), lambda i: (i, 0)),
        pl.BlockSpec((R, H), lambda i: (i, 0)),
        pl.BlockSpec((R, 1), lambda i: (i, 0)),
        pl.BlockSpec((1, D), lambda i: (0, 0)),
        pl.BlockSpec((D, D), lambda i: (0, 0)),
        pl.BlockSpec((1, D), lambda i: (0, 0)),
        pl.BlockSpec((D, 1), lambda i: (0, 0)),
        pl.BlockSpec((1, 1), lambda i: (0, 0)),
    ],
    out_specs=pl.BlockSpec((1, 1), lambda i: (0, 0)),
    out_shape=jax.ShapeDtypeStruct((1, 1), jnp.float32),
    scratch_shapes=[pltpu.VMEM((1, D), jnp.float32)],
)


def kernel(x, edge_index, W1, b1, W2, b2, Wl1, bl1, Wl2, bl2):
    src2 = edge_index[0].reshape(NCH, EC)
    dst2 = edge_index[1].reshape(NCH, EC)
    dga, dgb = _deg(dst2)
    y0, y1, u = _tc1(x, W1, dga[:N].reshape(N, 1), dgb[:N].reshape(N, 1))
    z0, z1 = _edge(y0, y1, src2, dst2)
    y0, y1 = _tc2(z0, z1, u, b1.reshape(1, D), W2)
    z0, z1 = _edge(y0, y1, src2, dst2)
    return _tc3(z0, z1, u, b2.reshape(1, D), Wl1, bl1.reshape(1, D),
                Wl2, bl2.reshape(1, 1))


# speedup vs baseline: 20.4563x; 1.0304x over previous
import jax
import jax.numpy as jnp
from jax import lax
from jax.experimental import pallas as pl
from jax.experimental.pallas import tpu as pltpu
from jax.experimental.pallas import tpu_sc as plsc

N = 10000
NP = 10240
E = 160000
D = 256
H = 128
R = 2000
NT = 16
STRIPE = NP // NT
EC = 125
NCH = E // EC
CPT = NCH // NT
HB = CPT // 2
DPT = NCH // 2 // NT

_sc_mesh = plsc.VectorSubcoreMesh(core_axis_name="c", subcore_axis_name="s")


def _stripe_copy(src_ref, dst_ref, s):
    pltpu.sync_copy(src_ref.at[pl.ds(s * STRIPE, STRIPE)],
                    dst_ref.at[pl.ds(s * STRIPE, STRIPE)])


def _deg_body(dst2, dga, dgb, idx_d, ones_v, zbuf, acc):
    c = lax.axis_index("c")
    s = lax.axis_index("s")
    for off in range(0, 128, 16):
        ones_v[pl.ds(off, 16)] = jnp.ones((16,), jnp.float32)

    def _zb(k, carry):
        zbuf[pl.ds(k * 16, 16)] = jnp.zeros((16,), jnp.float32)
        return carry

    lax.fori_loop(0, STRIPE // 16, _zb, 0)

    pltpu.sync_copy(zbuf, acc.at[pl.ds(s * STRIPE, STRIPE)])
    plsc.subcore_barrier()

    for cc, out in ((0, dga), (1, dgb)):
        @pl.when(c == cc)
        def _(cc=cc, out=out):
            pltpu.sync_copy(dst2.at[pl.ds(cc * (NCH // 2) + s * DPT, DPT)],
                            idx_d)

            def _body(j, carry):
                pltpu.sync_copy(ones_v.at[pl.ds(0, EC)], acc.at[idx_d.at[j]],
                                add=True)
                return carry

            lax.fori_loop(0, DPT, _body, 0)
            plsc.subcore_barrier()
            _stripe_copy(acc, out, s)


_deg = pl.kernel(
    _deg_body,
    out_type=[jax.ShapeDtypeStruct((NP,), jnp.float32),
              jax.ShapeDtypeStruct((NP,), jnp.float32)],
    mesh=_sc_mesh,
    scratch_types=[
        pltpu.VMEM((DPT, EC), jnp.int32),
        pltpu.VMEM((128,), jnp.float32),
        pltpu.VMEM((STRIPE,), jnp.float32),
        pltpu.VMEM_SHARED((NP,), jnp.float32),
    ],
)


def _edge_body(yp0, yp1, src2, dst2, z0, z1, idx_s, idx_d, buf0, buf1, acc,
               sem0, sem1, semi):
    c = lax.axis_index("c")
    s = lax.axis_index("s")
    for cc, yp, z in ((0, yp0, z0), (1, yp1, z1)):
        @pl.when(c == cc)
        def _(yp=yp, z=z):
            dinit = pltpu.async_copy(yp.at[pl.ds(s * STRIPE, STRIPE)],
                                     acc.at[pl.ds(s * STRIPE, STRIPE)], semi)

            for h in range(2):
                rb = s * CPT + h * HB
                pltpu.sync_copy(src2.at[pl.ds(rb, HB)], idx_s)
                pltpu.sync_copy(dst2.at[pl.ds(rb, HB)], idx_d)
                if h == 0:
                    dinit.wait()
                    plsc.subcore_barrier()
                pltpu.async_copy(yp.at[idx_s.at[0]], buf0, sem0)
                pltpu.async_copy(yp.at[idx_s.at[1]], buf1, sem1)

                def _body(k, carry):
                    pltpu.make_async_copy(yp.at[idx_s.at[0]], buf0, sem0).wait()
                    pltpu.sync_copy(buf0, acc.at[idx_d.at[2 * k]], add=True)

                    @pl.when(k < HB // 2 - 1)
                    def _():
                        pltpu.async_copy(yp.at[idx_s.at[2 * k + 2]], buf0, sem0)

                    pltpu.make_async_copy(yp.at[idx_s.at[1]], buf1, sem1).wait()
                    pltpu.sync_copy(buf1, acc.at[idx_d.at[2 * k + 1]], add=True)

                    @pl.when(k < HB // 2 - 1)
                    def _():
                        pltpu.async_copy(yp.at[idx_s.at[2 * k + 3]], buf1, sem1)

                    return carry

                lax.fori_loop(0, HB // 2, _body, 0)

            plsc.subcore_barrier()
            _stripe_copy(acc, z, s)


_edge = pl.kernel(
    _edge_body,
    out_type=[jax.ShapeDtypeStruct((NP, H), jnp.float32),
              jax.ShapeDtypeStruct((NP, H), jnp.float32)],
    mesh=_sc_mesh,
    scratch_types=[
        pltpu.VMEM((HB, EC), jnp.int32),
        pltpu.VMEM((HB, EC), jnp.int32),
        pltpu.VMEM((EC, H), jnp.float32),
        pltpu.VMEM((EC, H), jnp.float32),
        pltpu.VMEM_SHARED((NP, H), jnp.float32),
        pltpu.SemaphoreType.DMA,
        pltpu.SemaphoreType.DMA,
        pltpu.SemaphoreType.DMA,
    ],
)


def _tc1_body(x_ref, w_ref, da_ref, db_ref, y0_ref, y1_ref, u_ref):
    u = lax.rsqrt(da_ref[...] + db_ref[...] + 1.0)
    y = jnp.dot(x_ref[...], w_ref[...], preferred_element_type=jnp.float32) * u
    y0_ref[...] = y[:, :H]
    y1_ref[...] = y[:, H:]
    u_ref[...] = u


_tc1 = pl.pallas_call(
    _tc1_body,
    grid=(N // R,),
    in_specs=[
        pl.BlockSpec((R, D), lambda i: (i, 0)),
        pl.BlockSpec((D, D), lambda i: (0, 0)),
        pl.BlockSpec((R, 1), lambda i: (i, 0)),
        pl.BlockSpec((R, 1), lambda i: (i, 0)),
    ],
    out_specs=[
        pl.BlockSpec((R, H), lambda i: (i, 0)),
        pl.BlockSpec((R, H), lambda i: (i, 0)),
        pl.BlockSpec((R, 1), lambda i: (i, 0)),
    ],
    out_shape=[
        jax.ShapeDtypeStruct((NP, H), jnp.float32),
        jax.ShapeDtypeStruct((NP, H), jnp.float32),
        jax.ShapeDtypeStruct((N, 1), jnp.float32),
    ],
)


def _tc2_body(z0_ref, z1_ref, u_ref, b_ref, w_ref, y0_ref, y1_ref):
    u = u_ref[...]
    h0 = jnp.maximum(z0_ref[...] * u + b_ref[:, :H], 0.0)
    h1 = jnp.maximum(z1_ref[...] * u + b_ref[:, H:], 0.0)
    h = jnp.concatenate([h0, h1], axis=1)
    y = jnp.dot(h, w_ref[...], preferred_element_type=jnp.float32) * u
    y0_ref[...] = y[:, :H]
    y1_ref[...] = y[:, H:]


_tc2 = pl.pallas_call(
    _tc2_body,
    grid=(N // R,),
    in_specs=[
        pl.BlockSpec((R, H), lambda i: (i, 0)),
        pl.BlockSpec((R, H), lambda i: (i, 0)),
        pl.BlockSpec((R, 1), lambda i: (i, 0)),
        pl.BlockSpec((1, D), lambda i: (0, 0)),
        pl.BlockSpec((D, D), lambda i: (0, 0)),
    ],
    out_specs=[
        pl.BlockSpec((R, H), lambda i: (i, 0)),
        pl.BlockSpec((R, H), lambda i: (i, 0)),
    ],
    out_shape=[
        jax.ShapeDtypeStruct((NP, H), jnp.float32),
        jax.ShapeDtypeStruct((NP, H), jnp.float32),
    ],
)


def _tc3_body(z0_ref, z1_ref, u_ref, b_ref, wl1_ref, bl1_ref, wl2_ref, bl2_ref,
              o_ref, acc_ref):
    i = pl.program_id(0)

    @pl.when(i == 0)
    def _():
        acc_ref[...] = jnp.zeros_like(acc_ref)

    u = u_ref[...]
    h0 = jnp.maximum(z0_ref[...] * u + b_ref[:, :H], 0.0)
    h1 = jnp.maximum(z1_ref[...] * u + b_ref[:, H:], 0.0)
    acc_ref[:, :H] += jnp.sum(h0, axis=0, keepdims=True)
    acc_ref[:, H:] += jnp.sum(h1, axis=0, keepdims=True)

    @pl.when(i == pl.num_programs(0) - 1)
    def _():
        g = acc_ref[...] * (1.0 / N)
        t = jnp.maximum(
            jnp.dot(g, wl1_ref[...], preferred_element_type=jnp.float32)
            + bl1_ref[...], 0.0)
        o = jnp.maximum(
            jnp.dot(t, wl2_ref[...], preferred_element_type=jnp.float32)
            + bl2_ref[...], 0.0)
        o_ref[...] = o


_tc3 = pl.pallas_call(
    _tc3_body,
    grid=(N // R,),
    in_specs=[
        pl.BlockSpec((R, H), lambda i: (i, 0)),
        pl.BlockSpec((R, H), lambda i: (i, 0)),
        pl.BlockSpec((R, 1), lambda i: (i, 0)),
        pl.BlockSpec((1, D), lambda i: (0, 0)),
        pl.BlockSpec((D, D), lambda i: (0, 0)),
        pl.BlockSpec((1, D), lambda i: (0, 0)),
        pl.BlockSpec((D, 1), lambda i: (0, 0)),
        pl.BlockSpec((1, 1), lambda i: (0, 0)),
    ],
    out_specs=pl.BlockSpec((1, 1), lambda i: (0, 0)),
    out_shape=jax.ShapeDtypeStruct((1, 1), jnp.float32),
    scratch_shapes=[pltpu.VMEM((1, D), jnp.float32)],
)


def kernel(x, edge_index, W1, b1, W2, b2, Wl1, bl1, Wl2, bl2):
    src2 = edge_index[0].reshape(NCH, EC)
    dst2 = edge_index[1].reshape(NCH, EC)
    dga, dgb = _deg(dst2)
    y0, y1, u = _tc1(x, W1, dga[:N].reshape(N, 1), dgb[:N].reshape(N, 1))
    z0, z1 = _edge(y0, y1, src2, dst2)
    y0, y1 = _tc2(z0, z1, u, b1.reshape(1, D), W2)
    z0, z1 = _edge(y0, y1, src2, dst2)
    return _tc3(z0, z1, u, b2.reshape(1, D), Wl1, bl1.reshape(1, D),
                Wl2, bl2.reshape(1, 1))
